# baseline (speedup 1.0000x reference)
import functools

import jax
import jax.numpy as jnp
from jax import lax
from jax.experimental import pallas as pl
from jax.experimental.pallas import tpu as pltpu
from jax.experimental.pallas import tpu_sc as plsc


def _rsqrt(x):
    i = plsc.bitcast(x, jnp.int32)
    i = jnp.int32(0x5F3759DF) - (i >> 1)
    y = plsc.bitcast(i, jnp.float32)
    xh = x * jnp.float32(0.5)
    for _ in range(3):
        y = y * (jnp.float32(1.5) - xh * y * y)
    return y


def kernel(ind, rots_emb_weight):
    B = ind.shape[0]
    V, D = rots_emb_weight.shape
    ind = ind.astype(jnp.int32)

    info = plsc.get_sparse_core_info()
    NC, NS, L = info.num_cores, info.num_subcores, info.num_lanes
    NW = NC * NS
    assert B % (8 * NW) == 0
    bpw = B // NW

    mesh = plsc.VectorSubcoreMesh(core_axis_name="c", subcore_axis_name="s")

    @functools.partial(
        pl.kernel,
        mesh=mesh,
        compiler_params=pltpu.CompilerParams(
            needs_layout_passes=False,
            use_tc_tiling_on_sc=False,
            disable_bounds_checks=True,
        ),
        out_type=jax.ShapeDtypeStruct((3, 3, B), jnp.float32),
        scratch_types=[
            [pltpu.VMEM((bpw,), jnp.int32) for _ in range(6)],
            [pltpu.VMEM((bpw,), jnp.float32) for _ in range(6)],
            [pltpu.VMEM((1, 1, bpw), jnp.float32) for _ in range(9)],
            pltpu.SemaphoreType.DMA,
        ],
    )
    def sc_kernel(ind_hbm, wt_hbm, out_hbm, idxs, cols, outs_v, sem):
        wid = lax.axis_index("s") * NC + lax.axis_index("c")
        base = wid * bpw
        pltpu.sync_copy(ind_hbm.at[pl.ds(base, bpw)], idxs[0])

        def mkidx(i, carry):
            s = pl.ds(i * L, L)
            v = idxs[0][s]
            for j in range(1, 6):
                idxs[j][s] = v + jnp.int32(j * V)
            return carry

        lax.fori_loop(0, bpw // L, mkidx, 0)
        waits = [
            pltpu.async_copy(wt_hbm.at[0].at[idxs[j]], cols[j], sem)
            for j in range(6)
        ]
        for w in waits:
            w.wait()

        def body(i, carry):
            s = pl.ds(i * L, L)
            a1, a2, a3 = cols[0][s], cols[1][s], cols[2][s]
            b1, b2, b3 = cols[3][s], cols[4][s], cols[5][s]
            n1 = _rsqrt(a1 * a1 + a2 * a2 + a3 * a3)
            e1x, e1y, e1z = a1 * n1, a2 * n1, a3 * n1
            d = e1x * b1 + e1y * b2 + e1z * b3
            u2x, u2y, u2z = b1 - d * e1x, b2 - d * e1y, b3 - d * e1z
            n2 = _rsqrt(u2x * u2x + u2y * u2y + u2z * u2z)
            e2x, e2y, e2z = u2x * n2, u2y * n2, u2z * n2
            e3x = e1y * e2z - e1z * e2y
            e3y = e1z * e2x - e1x * e2z
            e3z = e1x * e2y - e1y * e2x
            es = (e1x, e1y, e1z, e2x, e2y, e2z, e3x, e3y, e3z)
            for k in range(9):
                outs_v[k][0, 0, s] = es[k]
            return carry

        lax.fori_loop(0, bpw // L, body, 0)
        for k in range(9):
            pltpu.sync_copy(
                outs_v[k],
                out_hbm.at[
                    pl.ds(k // 3, 1), pl.ds(k % 3, 1), pl.ds(base, bpw)
                ],
            )

    out = sc_kernel(ind, rots_emb_weight.T)
    return out.transpose(2, 0, 1)


# speedup vs baseline: 3.8003x; 3.8003x over previous
import functools

import jax
import jax.numpy as jnp
from jax import lax
from jax.experimental import pallas as pl
from jax.experimental.pallas import tpu as pltpu
from jax.experimental.pallas import tpu_sc as plsc


def _rsqrt(x):
    i = plsc.bitcast(x, jnp.int32)
    i = jnp.int32(0x5F3759DF) - (i >> 1)
    y = plsc.bitcast(i, jnp.float32)
    xh = x * jnp.float32(0.5)
    for _ in range(3):
        y = y * (jnp.float32(1.5) - xh * y * y)
    return y


def kernel(ind, rots_emb_weight):
    B = ind.shape[0]
    V, D = rots_emb_weight.shape
    ind = ind.astype(jnp.int32)

    info = plsc.get_sparse_core_info()
    NC, NS, L = info.num_cores, info.num_subcores, info.num_lanes
    NW = NC * NS
    assert B % (8 * NW) == 0
    bpw = B // NW

    mesh = plsc.VectorSubcoreMesh(core_axis_name="c", subcore_axis_name="s")

    @functools.partial(
        pl.kernel,
        mesh=mesh,
        compiler_params=pltpu.CompilerParams(
            needs_layout_passes=False,
            use_tc_tiling_on_sc=False,
            disable_bounds_checks=True,
        ),
        out_type=jax.ShapeDtypeStruct((3, 3, B), jnp.float32),
        scratch_types=[
            [pltpu.VMEM((bpw,), jnp.int32) for _ in range(1)],
            [pltpu.VMEM((bpw,), jnp.float32) for _ in range(6)],
            [pltpu.VMEM((1, 1, bpw), jnp.float32) for _ in range(9)],
            pltpu.SemaphoreType.DMA,
        ],
    )
    def sc_kernel(ind_hbm, p0, p1, p2, p3, p4, p5, out_hbm, idxs, cols, outs_v, sem):
        planes = (p0, p1, p2, p3, p4, p5)
        wid = lax.axis_index("s") * NC + lax.axis_index("c")
        base = wid * bpw
        pltpu.sync_copy(ind_hbm.at[pl.ds(base, bpw)], idxs[0])
        waits = [
            pltpu.async_copy(planes[j].at[idxs[0]], cols[j], sem)
            for j in range(6)
        ]
        for w in waits:
            w.wait()

        def body(i, carry):
            s = pl.ds(i * L, L)
            a1, a2, a3 = cols[0][s], cols[1][s], cols[2][s]
            b1, b2, b3 = cols[3][s], cols[4][s], cols[5][s]
            n1 = _rsqrt(a1 * a1 + a2 * a2 + a3 * a3)
            e1x, e1y, e1z = a1 * n1, a2 * n1, a3 * n1
            d = e1x * b1 + e1y * b2 + e1z * b3
            u2x, u2y, u2z = b1 - d * e1x, b2 - d * e1y, b3 - d * e1z
            n2 = _rsqrt(u2x * u2x + u2y * u2y + u2z * u2z)
            e2x, e2y, e2z = u2x * n2, u2y * n2, u2z * n2
            e3x = e1y * e2z - e1z * e2y
            e3y = e1z * e2x - e1x * e2z
            e3z = e1x * e2y - e1y * e2x
            es = (e1x, e1y, e1z, e2x, e2y, e2z, e3x, e3y, e3z)
            for k in range(9):
                outs_v[k][0, 0, s] = es[k]
            return carry

        lax.fori_loop(0, bpw // L, body, 0)
        for k in range(9):
            pltpu.sync_copy(
                outs_v[k],
                out_hbm.at[
                    pl.ds(k // 3, 1), pl.ds(k % 3, 1), pl.ds(base, bpw)
                ],
            )

    out = sc_kernel(ind, *[rots_emb_weight[:, j] for j in range(D)])
    return out.transpose(2, 0, 1)


# speedup vs baseline: 9.8330x; 2.5874x over previous
import functools

import jax
import jax.numpy as jnp
from jax import lax
from jax.experimental import pallas as pl
from jax.experimental.pallas import tpu as pltpu
from jax.experimental.pallas import tpu_sc as plsc
from jax.experimental.layout import Format, Layout
from jax.experimental.layout import with_layout_constraint


def _rsqrt(x):
    i = plsc.bitcast(x, jnp.int32)
    i = jnp.int32(0x5F3759DF) - (i >> 1)
    y = plsc.bitcast(i, jnp.float32)
    xh = x * jnp.float32(0.5)
    for _ in range(3):
        y = y * (jnp.float32(1.5) - xh * y * y)
    return y


def kernel(ind, rots_emb_weight):
    B = ind.shape[0]
    V, D = rots_emb_weight.shape
    ind = ind.astype(jnp.int32)

    info = plsc.get_sparse_core_info()
    NC, NS, L = info.num_cores, info.num_subcores, info.num_lanes
    NW = NC * NS
    assert B % (8 * NW) == 0
    bpw = B // NW

    mesh = plsc.VectorSubcoreMesh(core_axis_name="c", subcore_axis_name="s")

    @functools.partial(
        pl.kernel,
        mesh=mesh,
        compiler_params=pltpu.CompilerParams(
            needs_layout_passes=False,
            use_tc_tiling_on_sc=False,
            disable_bounds_checks=True,
        ),
        out_type=jax.ShapeDtypeStruct((3, 3, B), jnp.float32),
        scratch_types=[
            [pltpu.VMEM((bpw,), jnp.int32) for _ in range(6)],
            [pltpu.VMEM((bpw,), jnp.float32) for _ in range(6)],
            [pltpu.VMEM((1, 1, bpw), jnp.float32) for _ in range(9)],
            pltpu.SemaphoreType.DMA,
        ],
    )
    def sc_kernel(ind_hbm, wt_hbm, out_hbm, idxs, cols, outs_v, sem):
        wid = lax.axis_index("s") * NC + lax.axis_index("c")
        base = wid * bpw
        pltpu.sync_copy(ind_hbm.at[pl.ds(base, bpw)], idxs[0])

        def mkidx(i, carry):
            s = pl.ds(i * L, L)
            v = idxs[0][s]
            pb = ((v >> 7) << 10) | (v & jnp.int32(127))
            for j in range(1, 6):
                idxs[j][s] = pb + jnp.int32(j * 128)
            idxs[0][s] = pb
            return carry

        lax.fori_loop(0, bpw // L, mkidx, 0)
        waits = [
            pltpu.async_copy(wt_hbm.at[0].at[idxs[j]], cols[j], sem)
            for j in range(6)
        ]
        for w in waits:
            w.wait()

        def body(i, carry):
            s = pl.ds(i * L, L)
            a1, a2, a3 = cols[0][s], cols[1][s], cols[2][s]
            b1, b2, b3 = cols[3][s], cols[4][s], cols[5][s]
            n1 = _rsqrt(a1 * a1 + a2 * a2 + a3 * a3)
            e1x, e1y, e1z = a1 * n1, a2 * n1, a3 * n1
            d = e1x * b1 + e1y * b2 + e1z * b3
            u2x, u2y, u2z = b1 - d * e1x, b2 - d * e1y, b3 - d * e1z
            n2 = _rsqrt(u2x * u2x + u2y * u2y + u2z * u2z)
            e2x, e2y, e2z = u2x * n2, u2y * n2, u2z * n2
            e3x = e1y * e2z - e1z * e2y
            e3y = e1z * e2x - e1x * e2z
            e3z = e1x * e2y - e1y * e2x
            es = (e1x, e1y, e1z, e2x, e2y, e2z, e3x, e3y, e3z)
            for k in range(9):
                outs_v[k][0, 0, s] = es[k]
            return carry

        lax.fori_loop(0, bpw // L, body, 0)
        for k in range(9):
            pltpu.sync_copy(
                outs_v[k],
                out_hbm.at[
                    pl.ds(k // 3, 1), pl.ds(k % 3, 1), pl.ds(base, bpw)
                ],
            )

    wt = with_layout_constraint(
        rots_emb_weight.T,
        Layout(major_to_minor=(0, 1), tiling=((8, 128),)),
    )
    out = sc_kernel(ind, wt)
    return out.transpose(2, 0, 1)
